# baseline (speedup 1.0000x reference)
import functools

import jax
import jax.numpy as jnp
from jax import lax
from jax.experimental import pallas as pl
from jax.experimental.pallas import tpu as pltpu
from jax.experimental.pallas import tpu_sc as plsc

_VOCAB = 100000
_D = 128
_HIST = 50
_BATCH = 4096
_B = _BATCH * _HIST
_NW = 32
_B_PER_W = _B // _NW
_CHUNK = 200
_NCHUNK = _B_PER_W // _CHUNK
_NBUF = 4


def _make_gather():
    mesh = plsc.VectorSubcoreMesh(core_axis_name="c", subcore_axis_name="s")

    @functools.partial(
        pl.kernel,
        mesh=mesh,
        out_type=jax.ShapeDtypeStruct((_B, _D), jnp.float32),
        scratch_types=(
            [pltpu.VMEM((_B_PER_W,), jnp.int32)]
            + [pltpu.VMEM((_CHUNK, _D), jnp.float32) for _ in range(_NBUF)]
            + [pltpu.SemaphoreType.DMA for _ in range(2 * _NBUF)]
        ),
    )
    def gather_kernel(idx_hbm, table_hbm, out_hbm, idx_v, *bufs):
        rows_v = bufs[:_NBUF]
        gsem = bufs[_NBUF:2 * _NBUF]
        wsem = bufs[2 * _NBUF:]
        wid = lax.axis_index("s") * 2 + lax.axis_index("c")
        base = wid * _B_PER_W

        pltpu.sync_copy(idx_hbm.at[pl.ds(base, _B_PER_W)], idx_v)

        ngroup = _NCHUNK // _NBUF

        def group(j, carry):
            goff = j * _NBUF * _CHUNK
            for b in range(_NBUF):
                def wait_write(b=b):
                    pltpu.make_async_copy(
                        rows_v[b],
                        out_hbm.at[pl.ds(base + goff + b * _CHUNK, _CHUNK)],
                        wsem[b]).wait()
                pl.when(j > 0)(wait_write)
                pltpu.async_copy(
                    table_hbm.at[idx_v.at[pl.ds(goff + b * _CHUNK, _CHUNK)]],
                    rows_v[b], gsem[b])
            for b in range(_NBUF):
                pltpu.make_async_copy(
                    table_hbm.at[idx_v.at[pl.ds(goff + b * _CHUNK, _CHUNK)]],
                    rows_v[b], gsem[b]).wait()
                pltpu.async_copy(
                    rows_v[b],
                    out_hbm.at[pl.ds(base + goff + b * _CHUNK, _CHUNK)],
                    wsem[b])
            return carry

        lax.fori_loop(0, ngroup, group, 0)
        for b in range(_NBUF):
            pltpu.make_async_copy(
                rows_v[b],
                out_hbm.at[pl.ds(base + (ngroup - 1) * _NBUF * _CHUNK
                                 + b * _CHUNK, _CHUNK)],
                wsem[b]).wait()

    return gather_kernel


_gather = _make_gather()


@jax.jit
def kernel(x, table):
    idx = jnp.transpose(x).reshape(-1).astype(jnp.int32)
    out = _gather(idx, table)
    return jnp.transpose(out.reshape(_HIST, _BATCH, _D), (1, 0, 2))


# speedup vs baseline: 1.0394x; 1.0394x over previous
import functools

import jax
import jax.numpy as jnp
from jax import lax
from jax.experimental import pallas as pl
from jax.experimental.pallas import tpu as pltpu
from jax.experimental.pallas import tpu_sc as plsc

_VOCAB = 100000
_D = 128
_HIST = 50
_BATCH = 4096
_B = _BATCH * _HIST
_NW = 32
_B_PER_W = _B // _NW
_CHUNK = 400
_NCHUNK = _B_PER_W // _CHUNK
_NBUF = 2


def _make_gather():
    mesh = plsc.VectorSubcoreMesh(core_axis_name="c", subcore_axis_name="s")

    @functools.partial(
        pl.kernel,
        mesh=mesh,
        out_type=jax.ShapeDtypeStruct((_B, _D), jnp.float32),
        scratch_types=(
            [pltpu.VMEM((_B_PER_W,), jnp.int32)]
            + [pltpu.VMEM((_CHUNK, _D), jnp.float32) for _ in range(_NBUF)]
            + [pltpu.SemaphoreType.DMA for _ in range(2 * _NBUF)]
        ),
    )
    def gather_kernel(idx_hbm, table_hbm, out_hbm, idx_v, *bufs):
        rows_v = bufs[:_NBUF]
        gsem = bufs[_NBUF:2 * _NBUF]
        wsem = bufs[2 * _NBUF:]
        wid = lax.axis_index("s") * 2 + lax.axis_index("c")
        base = wid * _B_PER_W

        pltpu.sync_copy(idx_hbm.at[pl.ds(base, _B_PER_W)], idx_v)

        gdesc = [None] * _NCHUNK
        wdesc = [None] * _NCHUNK

        def drain(ci):
            q = ci % _NBUF
            gdesc[ci].wait()
            wdesc[ci] = pltpu.async_copy(
                rows_v[q], out_hbm.at[pl.ds(base + ci * _CHUNK, _CHUNK)],
                wsem[q])

        for ci in range(_NCHUNK):
            p = ci % _NBUF
            if ci >= _NBUF:
                wdesc[ci - _NBUF].wait()
            gdesc[ci] = pltpu.async_copy(
                table_hbm.at[idx_v.at[pl.ds(ci * _CHUNK, _CHUNK)]],
                rows_v[p], gsem[p])
            if ci >= 1:
                drain(ci - 1)
        drain(_NCHUNK - 1)
        for ci in range(_NCHUNK - _NBUF, _NCHUNK):
            wdesc[ci].wait()

    return gather_kernel


_gather = _make_gather()


@jax.jit
def kernel(x, table):
    idx = jnp.transpose(x).reshape(-1).astype(jnp.int32)
    out = _gather(idx, table)
    return jnp.transpose(out.reshape(_HIST, _BATCH, _D), (1, 0, 2))
